# baseline (speedup 1.0000x reference)
import functools

import jax
import jax.numpy as jnp
import numpy as np
from jax import lax
from jax.experimental import pallas as pl
from jax.experimental.pallas import tpu as pltpu
from jax.experimental.pallas import tpu_sc as plsc

_BUCKET_SIZE = 512
_HX = np.int32(73856093)
_HY = np.int32(19349663)
_HZ = np.int32(83492791)
_HB = np.int32(-1640531527)
_DUMP = 2048


def _bucket_ids(x, y, z, pid, sep_scalars, multv, n_buckets):
  one = jnp.ones((16,), jnp.int32)
  zero = jnp.zeros((16,), jnp.int32)
  qx = x.astype(jnp.int32)
  qx = qx - jnp.where(qx.astype(jnp.float32) > x, one, zero)
  qy = y.astype(jnp.int32)
  qy = qy - jnp.where(qy.astype(jnp.float32) > y, one, zero)
  qz = z.astype(jnp.int32)
  qz = qz - jnp.where(qz.astype(jnp.float32) > z, one, zero)
  h = (qx * _HX) ^ (qy * _HY) ^ (qz * _HZ)
  batch = zero
  for s in sep_scalars:
    batch = batch + jnp.where(pid >= s, one, zero)
  h = h ^ (batch * multv)
  return h & jnp.int32(n_buckets - 1)


@functools.lru_cache(maxsize=None)
def _build(n, n_buckets, nsep, nc, ns):
  nw = nc * ns
  pts_w = n // nw
  chunk = 2048
  nchunk = pts_w // chunk
  vpc = chunk // 16
  bkt_w = n_buckets // nw
  pad_to = n
  mesh = plsc.VectorSubcoreMesh(core_axis_name="c", subcore_axis_name="s")
  cparams = pltpu.CompilerParams(needs_layout_passes=False)

  @functools.partial(
      pl.kernel,
      out_type=(
          jax.ShapeDtypeStruct((n,), jnp.int32),
          jax.ShapeDtypeStruct((nw, n_buckets), jnp.int32),
      ),
      mesh=mesh,
      compiler_params=cparams,
      scratch_types=[
          pltpu.VMEM((16,), jnp.int32),
          pltpu.VMEM((16,), jnp.int32),
          pltpu.VMEM((chunk,), jnp.float32),
          pltpu.VMEM((chunk,), jnp.float32),
          pltpu.VMEM((chunk,), jnp.float32),
          pltpu.VMEM((chunk,), jnp.int32),
          pltpu.VMEM((16 * n_buckets,), jnp.int32),
          pltpu.VMEM((n_buckets,), jnp.int32),
      ],
  )
  def k_hist(xs, ys, zs, sepsv, mv, bid_hbm, hist_hbm,
             sv, mvv, xc, yc, zc, bst, table, hred):
    cid = lax.axis_index("c")
    sid = lax.axis_index("s")
    w = sid * nc + cid
    base_w = w * pts_w
    iota = jnp.arange(16, dtype=jnp.int32)
    pltpu.sync_copy(sepsv, sv)
    pltpu.sync_copy(mv, mvv)
    svv = sv[...]
    seps = [svv[j] for j in range(nsep)]
    multv = mvv[...]

    @pl.loop(0, 16 * n_buckets // 16)
    def _(v):
      table[pl.ds(v * 16, 16)] = jnp.zeros((16,), jnp.int32)

    @pl.loop(0, nchunk)
    def _(c):
      off = base_w + c * chunk
      pltpu.sync_copy(xs.at[pl.ds(off, chunk)], xc)
      pltpu.sync_copy(ys.at[pl.ds(off, chunk)], yc)
      pltpu.sync_copy(zs.at[pl.ds(off, chunk)], zc)

      @pl.loop(0, vpc)
      def _(v):
        x = xc[pl.ds(v * 16, 16)]
        y = yc[pl.ds(v * 16, 16)]
        z = zc[pl.ds(v * 16, 16)]
        pid = off + v * 16 + iota
        b = _bucket_ids(x, y, z, pid, seps, multv, n_buckets)
        bst[pl.ds(v * 16, 16)] = b
        plsc.addupdate_scatter(table, [iota * n_buckets + b],
                               jnp.ones((16,), jnp.int32))

      pltpu.sync_copy(bst, bid_hbm.at[pl.ds(off, chunk)])

    @pl.loop(0, n_buckets // 16)
    def _(v):
      acc = jnp.zeros((16,), jnp.int32)
      for l in range(16):
        acc = acc + table[pl.ds(l * n_buckets + v * 16, 16)]
      hred[pl.ds(v * 16, 16)] = acc

    pltpu.sync_copy(hred, hist_hbm.at[w])

  @functools.partial(
      pl.kernel,
      out_type=(
          jax.ShapeDtypeStruct((n,), jnp.int32),
          jax.ShapeDtypeStruct((pad_to + _DUMP,), jnp.float32),
          jax.ShapeDtypeStruct((pad_to + _DUMP,), jnp.float32),
          jax.ShapeDtypeStruct((pad_to + _DUMP,), jnp.float32),
          jax.ShapeDtypeStruct((n_buckets,), jnp.int32),
      ),
      mesh=mesh,
      compiler_params=cparams,
      scratch_types=[
          pltpu.VMEM((nw, n_buckets), jnp.int32),
          pltpu.VMEM((n_buckets,), jnp.int32),
          pltpu.VMEM((n_buckets + 16,), jnp.int32),
          pltpu.VMEM((chunk,), jnp.int32),
          pltpu.VMEM((chunk,), jnp.float32),
          pltpu.VMEM((chunk,), jnp.float32),
          pltpu.VMEM((chunk,), jnp.float32),
          pltpu.VMEM((chunk,), jnp.int32),
          pltpu.VMEM((chunk,), jnp.int32),
          pltpu.SemaphoreType.DMA,
          pltpu.SemaphoreType.DMA,
          pltpu.SemaphoreType.DMA,
      ],
  )
  def k_scatter(bid_hbm, xs, ys, zs, hist_hbm,
                sidx_hbm, sx_hbm, sy_hbm, sz_hbm, bcnt_hbm,
                rows, cnts, tot, bidc, xc, yc, zc, sst, tst,
                sem1, sem2, sem3):
    cid = lax.axis_index("c")
    sid = lax.axis_index("s")
    w = sid * nc + cid
    base_w = w * pts_w
    iota = jnp.arange(16, dtype=jnp.int32)

    pltpu.sync_copy(hist_hbm, rows)

    @pl.loop(0, n_buckets // 16)
    def _(v):
      acc = jnp.zeros((16,), jnp.int32)

      def add_row(r, a):
        return a + rows[r, pl.ds(v * 16, 16)]

      acc = lax.fori_loop(0, w, add_row, acc)
      cnts[pl.ds(v * 16, 16)] = acc
      acc = lax.fori_loop(w, nw, add_row, acc)
      tot[pl.ds(v * 16, 16)] = acc

    pltpu.sync_copy(tot.at[pl.ds(w * bkt_w, bkt_w)],
                    bcnt_hbm.at[pl.ds(w * bkt_w, bkt_w)])

    @pl.loop(0, nchunk)
    def _(c):
      off = base_w + c * chunk
      pltpu.sync_copy(bid_hbm.at[pl.ds(off, chunk)], bidc)
      pltpu.sync_copy(xs.at[pl.ds(off, chunk)], xc)
      pltpu.sync_copy(ys.at[pl.ds(off, chunk)], yc)
      pltpu.sync_copy(zs.at[pl.ds(off, chunk)], zc)

      @pl.loop(0, vpc)
      def _(v):
        b = bidc[pl.ds(v * 16, 16)]
        cnt, lastm = plsc.scan_count(b)
        g = plsc.load_gather(cnts, [b])
        plsc.store_scatter(cnts, [b], g + cnt, mask=lastm)
        grank = g + cnt - 1
        valid = grank < _BUCKET_SIZE
        pos = b * _BUCKET_SIZE + grank
        sst[pl.ds(v * 16, 16)] = jnp.where(valid, pos, -1)
        dump = pad_to + (v % 128) * 16 + iota
        tst[pl.ds(v * 16, 16)] = jnp.where(valid, pos, dump)

      d1 = pltpu.async_copy(xc, sx_hbm.at[tst], sem1)
      d2 = pltpu.async_copy(yc, sy_hbm.at[tst], sem2)
      d3 = pltpu.async_copy(zc, sz_hbm.at[tst], sem3)
      d1.wait()
      d2.wait()
      d3.wait()
      pltpu.sync_copy(sst, sidx_hbm.at[pl.ds(off, chunk)])

  def _clean_body(x_ref, y_ref, z_ref, cnt_ref, ox_ref, oy_ref, oz_ref):
    rank = lax.broadcasted_iota(jnp.int32, (8, _BUCKET_SIZE), 1)
    mask = rank < cnt_ref[...]
    zero = jnp.zeros((8, _BUCKET_SIZE), jnp.float32)
    ox_ref[...] = jnp.where(mask, x_ref[...], zero)
    oy_ref[...] = jnp.where(mask, y_ref[...], zero)
    oz_ref[...] = jnp.where(mask, z_ref[...], zero)

  plane_spec = pl.BlockSpec((8, _BUCKET_SIZE), lambda i: (i, 0))
  cnt_spec = pl.BlockSpec((8, 1), lambda i: (i, 0))
  k_clean = pl.pallas_call(
      _clean_body,
      grid=(n_buckets // 8,),
      in_specs=[plane_spec, plane_spec, plane_spec, cnt_spec],
      out_specs=[plane_spec, plane_spec, plane_spec],
      out_shape=[
          jax.ShapeDtypeStruct((n_buckets, _BUCKET_SIZE), jnp.float32),
          jax.ShapeDtypeStruct((n_buckets, _BUCKET_SIZE), jnp.float32),
          jax.ShapeDtypeStruct((n_buckets, _BUCKET_SIZE), jnp.float32),
      ],
  )

  return k_hist, k_scatter, k_clean


def kernel(coords, seps, hash_op):
  n = coords.shape[0]
  bs = _BUCKET_SIZE
  pad_to = ((n + bs - 1) // bs) * bs
  n_buckets = pad_to // bs
  nsep = seps.shape[0]
  info = plsc.get_sparse_core_info()
  nc, ns = info.num_cores, info.num_subcores

  xs = coords[:, 0]
  ys = coords[:, 1]
  zs = coords[:, 2]
  seps16 = jnp.full((16,), np.int32(2**31 - 1), jnp.int32)
  seps16 = seps16.at[:nsep].set(seps.astype(jnp.int32))
  mult = jnp.where(jnp.asarray(hash_op) != 0, _HB, np.int32(0))
  mult16 = jnp.broadcast_to(mult.astype(jnp.int32), (16,))

  k_hist, k_scatter, k_clean = _build(n, n_buckets, nsep, nc, ns)
  bid, hist = k_hist(xs, ys, zs, seps16, mult16)
  sidx, sx, sy, sz, bcnt = k_scatter(bid, xs, ys, zs, hist)

  nrow = n_buckets + _DUMP // bs
  cx, cy, cz = k_clean(sx.reshape(nrow, bs), sy.reshape(nrow, bs),
                       sz.reshape(nrow, bs), bcnt.reshape(n_buckets, 1))
  scattered = jnp.stack(
      [cx.reshape(pad_to), cy.reshape(pad_to), cz.reshape(pad_to)], axis=-1)
  return scattered, sidx, bcnt


# speedup vs baseline: 1.9711x; 1.9711x over previous
import functools

import jax
import jax.numpy as jnp
import numpy as np
from jax import lax
from jax.experimental import pallas as pl
from jax.experimental.pallas import tpu as pltpu
from jax.experimental.pallas import tpu_sc as plsc

_BUCKET_SIZE = 512
_HX = np.int32(73856093)
_HY = np.int32(19349663)
_HZ = np.int32(83492791)
_HB = np.int32(-1640531527)
_DUMP = 2048


def _bucket_ids(x, y, z, pid, sep_scalars, multv, n_buckets):
  one = jnp.ones((16,), jnp.int32)
  zero = jnp.zeros((16,), jnp.int32)
  qx = x.astype(jnp.int32)
  qx = qx - jnp.where(qx.astype(jnp.float32) > x, one, zero)
  qy = y.astype(jnp.int32)
  qy = qy - jnp.where(qy.astype(jnp.float32) > y, one, zero)
  qz = z.astype(jnp.int32)
  qz = qz - jnp.where(qz.astype(jnp.float32) > z, one, zero)
  h = (qx * _HX) ^ (qy * _HY) ^ (qz * _HZ)
  batch = zero
  for s in sep_scalars:
    batch = batch + jnp.where(pid >= s, one, zero)
  h = h ^ (batch * multv)
  return h & jnp.int32(n_buckets - 1)


@functools.lru_cache(maxsize=None)
def _build(n, n_buckets, nsep, nc, ns):
  nw = nc * ns
  pts_w = n // nw
  chunk = 2048
  nchunk = pts_w // chunk
  vpc = chunk // 16
  bkt_w = n_buckets // nw
  pad_to = n
  mesh = plsc.VectorSubcoreMesh(core_axis_name="c", subcore_axis_name="s")
  cparams = pltpu.CompilerParams(needs_layout_passes=False)

  @functools.partial(
      pl.kernel,
      out_type=(
          jax.ShapeDtypeStruct((n,), jnp.int32),
          jax.ShapeDtypeStruct((nw, n_buckets), jnp.int32),
      ),
      mesh=mesh,
      compiler_params=cparams,
      scratch_types=[
          pltpu.VMEM((16,), jnp.int32),
          pltpu.VMEM((16,), jnp.int32),
          pltpu.VMEM((chunk,), jnp.float32),
          pltpu.VMEM((chunk,), jnp.float32),
          pltpu.VMEM((chunk,), jnp.float32),
          pltpu.VMEM((chunk,), jnp.int32),
          pltpu.VMEM((16 * n_buckets,), jnp.int32),
          pltpu.VMEM((n_buckets,), jnp.int32),
      ],
  )
  def k_hist(xs, ys, zs, sepsv, mv, bid_hbm, hist_hbm,
             sv, mvv, xc, yc, zc, bst, table, hred):
    cid = lax.axis_index("c")
    sid = lax.axis_index("s")
    w = sid * nc + cid
    base_w = w * pts_w
    iota = jnp.arange(16, dtype=jnp.int32)
    pltpu.sync_copy(sepsv, sv)
    pltpu.sync_copy(mv, mvv)
    svv = sv[...]
    seps = [svv[j] for j in range(nsep)]
    multv = mvv[...]

    @pl.loop(0, 16 * n_buckets // 16)
    def _(v):
      table[pl.ds(v * 16, 16)] = jnp.zeros((16,), jnp.int32)

    @pl.loop(0, nchunk)
    def _(c):
      off = base_w + c * chunk
      pltpu.sync_copy(xs.at[pl.ds(off, chunk)], xc)
      pltpu.sync_copy(ys.at[pl.ds(off, chunk)], yc)
      pltpu.sync_copy(zs.at[pl.ds(off, chunk)], zc)

      @pl.loop(0, vpc)
      def _(v):
        x = xc[pl.ds(v * 16, 16)]
        y = yc[pl.ds(v * 16, 16)]
        z = zc[pl.ds(v * 16, 16)]
        pid = off + v * 16 + iota
        b = _bucket_ids(x, y, z, pid, seps, multv, n_buckets)
        bst[pl.ds(v * 16, 16)] = b
        plsc.addupdate_scatter(table, [iota * n_buckets + b],
                               jnp.ones((16,), jnp.int32))

      pltpu.sync_copy(bst, bid_hbm.at[pl.ds(off, chunk)])

    @pl.loop(0, n_buckets // 16)
    def _(v):
      acc = jnp.zeros((16,), jnp.int32)
      for l in range(16):
        acc = acc + table[pl.ds(l * n_buckets + v * 16, 16)]
      hred[pl.ds(v * 16, 16)] = acc

    pltpu.sync_copy(hred, hist_hbm.at[w])

  @functools.partial(
      pl.kernel,
      out_type=(
          jax.ShapeDtypeStruct((n,), jnp.int32),
          jax.ShapeDtypeStruct((pad_to + 64, 16), jnp.float32),
          jax.ShapeDtypeStruct((n_buckets,), jnp.int32),
      ),
      mesh=mesh,
      compiler_params=pltpu.CompilerParams(needs_layout_passes=False,
                                           use_tc_tiling_on_sc=False),
      scratch_types=[
          pltpu.VMEM((nw, n_buckets), jnp.int32),
          pltpu.VMEM((n_buckets,), jnp.int32),
          pltpu.VMEM((n_buckets + 16,), jnp.int32),
          pltpu.VMEM((chunk,), jnp.int32),
          pltpu.VMEM((chunk,), jnp.float32),
          pltpu.VMEM((chunk,), jnp.float32),
          pltpu.VMEM((chunk,), jnp.float32),
          pltpu.VMEM((chunk,), jnp.int32),
          pltpu.VMEM((chunk,), jnp.int32),
          pltpu.VMEM((chunk, 16), jnp.float32),
          pltpu.SemaphoreType.DMA,
      ],
  )
  def k_scatter(bid_hbm, xs, ys, zs, hist_hbm,
                sidx_hbm, wide_hbm, bcnt_hbm,
                rows, cnts, tot, bidc, xc, yc, zc, sst, tst, stage,
                sem1):
    cid = lax.axis_index("c")
    sid = lax.axis_index("s")
    w = sid * nc + cid
    base_w = w * pts_w
    iota = jnp.arange(16, dtype=jnp.int32)

    pltpu.sync_copy(hist_hbm, rows)

    @pl.loop(0, n_buckets // 16)
    def _(v):
      acc = jnp.zeros((16,), jnp.int32)

      def add_row(r, a):
        return a + rows[r, pl.ds(v * 16, 16)]

      acc = lax.fori_loop(0, w, add_row, acc)
      cnts[pl.ds(v * 16, 16)] = acc
      acc = lax.fori_loop(w, nw, add_row, acc)
      tot[pl.ds(v * 16, 16)] = acc

    pltpu.sync_copy(tot.at[pl.ds(w * bkt_w, bkt_w)],
                    bcnt_hbm.at[pl.ds(w * bkt_w, bkt_w)])

    @pl.loop(0, nchunk)
    def _(c):
      off = base_w + c * chunk
      pltpu.sync_copy(bid_hbm.at[pl.ds(off, chunk)], bidc)
      pltpu.sync_copy(xs.at[pl.ds(off, chunk)], xc)
      pltpu.sync_copy(ys.at[pl.ds(off, chunk)], yc)
      pltpu.sync_copy(zs.at[pl.ds(off, chunk)], zc)

      @pl.loop(0, vpc)
      def _(v):
        b = bidc[pl.ds(v * 16, 16)]
        cnt, lastm = plsc.scan_count(b)
        g = plsc.load_gather(cnts, [b])
        plsc.store_scatter(cnts, [b], g + cnt, mask=lastm)
        grank = g + cnt - 1
        valid = grank < _BUCKET_SIZE
        pos = b * _BUCKET_SIZE + grank
        sst[pl.ds(v * 16, 16)] = jnp.where(valid, pos, -1)
        dump = pad_to + ((v * 16 + iota) & 63)
        tst[pl.ds(v * 16, 16)] = jnp.where(valid, pos, dump)
        rowv = v * 16 + iota
        plsc.store_scatter(stage, [rowv, iota * 0], xc[pl.ds(v * 16, 16)])
        plsc.store_scatter(stage, [rowv, iota * 0 + 1], yc[pl.ds(v * 16, 16)])
        plsc.store_scatter(stage, [rowv, iota * 0 + 2], zc[pl.ds(v * 16, 16)])

      pltpu.async_copy(stage, wide_hbm.at[tst], sem1).wait()
      pltpu.sync_copy(sst, sidx_hbm.at[pl.ds(off, chunk)])

  rows_b = _BUCKET_SIZE // 8

  def _clean_body(x_ref, cnt_ref, o_ref):
    r0 = lax.broadcasted_iota(jnp.int32, (rows_b, 128), 0)
    r1 = lax.broadcasted_iota(jnp.int32, (rows_b, 128), 1)
    rank = r0 * 8 + jnp.right_shift(r1, 4)
    mask = rank < cnt_ref[...].reshape(1, 1)
    o_ref[...] = jnp.where(mask, x_ref[...], jnp.zeros_like(rank, jnp.float32))

  k_clean = pl.pallas_call(
      _clean_body,
      grid=(n_buckets,),
      in_specs=[pl.BlockSpec((rows_b, 128), lambda i: (i, 0)),
                pl.BlockSpec((1, 1, 1), lambda i: (i, 0, 0))],
      out_specs=pl.BlockSpec((rows_b, 128), lambda i: (i, 0)),
      out_shape=jax.ShapeDtypeStruct((n_buckets * rows_b, 128), jnp.float32),
  )

  return k_hist, k_scatter, k_clean


def kernel(coords, seps, hash_op):
  n = coords.shape[0]
  bs = _BUCKET_SIZE
  pad_to = ((n + bs - 1) // bs) * bs
  n_buckets = pad_to // bs
  nsep = seps.shape[0]
  info = plsc.get_sparse_core_info()
  nc, ns = info.num_cores, info.num_subcores

  xs = coords[:, 0]
  ys = coords[:, 1]
  zs = coords[:, 2]
  seps16 = jnp.full((16,), np.int32(2**31 - 1), jnp.int32)
  seps16 = seps16.at[:nsep].set(seps.astype(jnp.int32))
  mult = jnp.where(jnp.asarray(hash_op) != 0, _HB, np.int32(0))
  mult16 = jnp.broadcast_to(mult.astype(jnp.int32), (16,))

  k_hist, k_scatter, k_clean = _build(n, n_buckets, nsep, nc, ns)
  bid, hist = k_hist(xs, ys, zs, seps16, mult16)
  sidx, wide, bcnt = k_scatter(bid, xs, ys, zs, hist)

  wide128 = wide.reshape((pad_to + 64) * 16 // 128, 128)
  clean = k_clean(wide128, bcnt.reshape(n_buckets, 1, 1))
  scattered = clean.reshape(pad_to, 16)[:, :3]
  return scattered, sidx, bcnt
